# baseline (device time: 352523 ns/iter reference)
import jax
import jax.numpy as jnp
from jax import lax
from jax.experimental import pallas as pl
from jax.experimental.pallas import tpu as pltpu

N_DEV = 4


def kernel(A, B):
    M = A.shape[0]
    N = B.shape[1]
    CH = M // N_DEV
    H = N // 2
    TJ = 1024
    NSUB = H // TJ

    A16 = A.astype(jnp.bfloat16)
    B16 = B.astype(jnp.bfloat16)

    def body(a_ref, b_ref, out_ref, commR, commL, p_ref,
             rs_send, rs_recv, ag_send, ag_recv, copy_sems):
        my = lax.axis_index("i")
        right = lax.rem(my + 1, N_DEV)
        left = lax.rem(my + N_DEV - 1, N_DEV)

        barrier = pltpu.get_barrier_semaphore()
        for nbr in (left, right):
            pl.semaphore_signal(barrier, inc=1, device_id=(nbr,),
                                device_id_type=pl.DeviceIdType.MESH)
        pl.semaphore_wait(barrier, 2)

        def a_blk(c):
            return a_ref[pl.ds(c * CH, CH), :]

        def mm_tile(c, col0):
            return jnp.dot(a_blk(c), b_ref[:, pl.ds(col0, TJ)],
                           preferred_element_type=jnp.float32
                           ).astype(jnp.bfloat16)

        def precompute(cR, cL):
            p_ref[:, pl.ds(0, H)] = jnp.dot(
                a_blk(cR), b_ref[:, pl.ds(0, H)],
                preferred_element_type=jnp.float32).astype(jnp.bfloat16)
            p_ref[:, pl.ds(H, H)] = jnp.dot(
                a_blk(cL), b_ref[:, pl.ds(H, H)],
                preferred_element_type=jnp.float32).astype(jnp.bfloat16)

        def add_sub(comm, slot, k, p_col0):
            col = pl.ds(k * TJ, TJ)
            pcol = pl.ds(p_col0 + k * TJ, TJ)
            comm[slot, :, col] = (
                comm[slot, :, col].astype(jnp.float32)
                + p_ref[:, pcol].astype(jnp.float32)
            ).astype(jnp.bfloat16)

        def rs_desc(s, k, comm, sub, dev):
            return pltpu.make_async_remote_copy(
                src_ref=comm.at[s % 2, :, pl.ds(k * TJ, TJ)],
                dst_ref=comm.at[(s + 1) % 2, :, pl.ds(k * TJ, TJ)],
                send_sem=rs_send.at[s, sub, k],
                recv_sem=rs_recv.at[s, sub, k],
                device_id=(dev,), device_id_type=pl.DeviceIdType.MESH,
            )

        rsR = [[rs_desc(s, k, commR, 0, right) for k in range(NSUB)]
               for s in range(N_DEV - 1)]
        rsL = [[rs_desc(s, k, commL, 1, left) for k in range(NSUB)]
               for s in range(N_DEV - 1)]

        def ag_desc(h, k, comm, sub, dev):
            return pltpu.make_async_remote_copy(
                src_ref=comm.at[(h + 1) % 2, :, pl.ds(k * TJ, TJ)],
                dst_ref=comm.at[h % 2, :, pl.ds(k * TJ, TJ)],
                send_sem=ag_send.at[h, sub, k],
                recv_sem=ag_recv.at[h, sub, k],
                device_id=(dev,), device_id_type=pl.DeviceIdType.MESH,
            )

        agR = [[ag_desc(h, k, commR, 0, right) for k in range(NSUB)]
               for h in range(N_DEV - 1)]
        agL = [[ag_desc(h, k, commL, 1, left) for k in range(NSUB)]
               for h in range(N_DEV - 1)]

        copies = []

        def store_half(comm, slot, c, col0, sem_idx):
            cp = pltpu.make_async_copy(
                comm.at[slot],
                out_ref.at[pl.ds(c * CH, CH), pl.ds(col0, H)],
                copy_sems.at[sem_idx],
            )
            cp.start()
            copies.append(cp)

        for k in range(NSUB):
            commR[0, :, pl.ds(k * TJ, TJ)] = mm_tile(my, k * TJ)
            rsR[0][k].start()
            commL[0, :, pl.ds(k * TJ, TJ)] = mm_tile(my, H + k * TJ)
            rsL[0][k].start()

        precompute(lax.rem(my - 1 + N_DEV, N_DEV), lax.rem(my + 1, N_DEV))

        for s in range(N_DEV - 1):
            last = s == N_DEV - 2
            for k in range(NSUB):
                rsR[s][k].wait()
                add_sub(commR, (s + 1) % 2, k, 0)
                if not last:
                    rsR[s + 1][k].start()
                else:
                    agR[0][k].start()
                    if k == NSUB - 1:
                        store_half(commR, 1, lax.rem(my + 1, N_DEV), 0, 0)
                rsL[s][k].wait()
                add_sub(commL, (s + 1) % 2, k, H)
                if not last:
                    rsL[s + 1][k].start()
                else:
                    agL[0][k].start()
                    if k == NSUB - 1:
                        store_half(commL, 1, lax.rem(my + 3, N_DEV), H, 1)
            if not last:
                precompute(lax.rem(my - s - 2 + N_DEV, N_DEV),
                           lax.rem(my + s + 2, N_DEV))

        for h in range(N_DEV - 1):
            for k in range(NSUB):
                agR[h][k].wait()
                if h < N_DEV - 2:
                    agR[h + 1][k].start()
                if k == NSUB - 1:
                    store_half(commR, h % 2, lax.rem(my - h + N_DEV, N_DEV),
                               0, 2 + 2 * h)
                agL[h][k].wait()
                if h < N_DEV - 2:
                    agL[h + 1][k].start()
                if k == NSUB - 1:
                    store_half(commL, h % 2, lax.rem(my + h, N_DEV),
                               H, 3 + 2 * h)

        for cp in copies:
            cp.wait()

    return pl.pallas_call(
        body,
        out_shape=jax.ShapeDtypeStruct((M, N), jnp.bfloat16),
        in_specs=[
            pl.BlockSpec(memory_space=pltpu.VMEM),
            pl.BlockSpec(memory_space=pltpu.VMEM),
        ],
        out_specs=pl.BlockSpec(memory_space=pl.ANY),
        scratch_shapes=[
            pltpu.VMEM((2, CH, H), jnp.bfloat16),
            pltpu.VMEM((2, CH, H), jnp.bfloat16),
            pltpu.VMEM((CH, N), jnp.bfloat16),
            pltpu.SemaphoreType.DMA((N_DEV - 1, 2, 2)),
            pltpu.SemaphoreType.DMA((N_DEV - 1, 2, 2)),
            pltpu.SemaphoreType.DMA((N_DEV - 1, 2, 2)),
            pltpu.SemaphoreType.DMA((N_DEV - 1, 2, 2)),
            pltpu.SemaphoreType.DMA((8,)),
        ],
        compiler_params=pltpu.CompilerParams(
            collective_id=0,
            vmem_limit_bytes=62 * 1024 * 1024,
        ),
    )(A16, B16)


# device time: 352068 ns/iter; 1.0013x vs baseline; 1.0013x over previous
import jax
import jax.numpy as jnp
from jax import lax
from jax.experimental import pallas as pl
from jax.experimental.pallas import tpu as pltpu

N_DEV = 4
NG = 2


def kernel(A, B):
    M = A.shape[0]
    N = B.shape[1]
    CH = M // N_DEV
    GR = CH // NG
    H = N // 2

    A16 = A.astype(jnp.bfloat16)
    B16 = B.astype(jnp.bfloat16)

    def body(a_ref, b_ref, out_ref, commR0, commR1, commL0, commL1,
             p0_ref, p1_ref, rs_send, rs_recv, ag_send, ag_recv, copy_sems):
        my = lax.axis_index("i")
        right = lax.rem(my + 1, N_DEV)
        left = lax.rem(my + N_DEV - 1, N_DEV)

        commR = [commR0, commR1]
        commL = [commL0, commL1]
        p_ref = [p0_ref, p1_ref]

        barrier = pltpu.get_barrier_semaphore()
        for nbr in (left, right):
            pl.semaphore_signal(barrier, inc=1, device_id=(nbr,),
                                device_id_type=pl.DeviceIdType.MESH)
        pl.semaphore_wait(barrier, 2)

        def a_blk(c, g):
            return a_ref[pl.ds(c * CH + g * GR, GR), :]

        def mm_half(c, g, col0):
            return jnp.dot(a_blk(c, g), b_ref[:, pl.ds(col0, H)],
                           preferred_element_type=jnp.float32
                           ).astype(jnp.bfloat16)

        def precompute(g, cR, cL):
            p_ref[g][:, pl.ds(0, H)] = mm_half(cR, g, 0)
            p_ref[g][:, pl.ds(H, H)] = mm_half(cL, g, H)

        def add_staged(g, comm, slot, p_col0):
            comm[slot] = (
                comm[slot].astype(jnp.float32)
                + p_ref[g][:, pl.ds(p_col0, H)].astype(jnp.float32)
            ).astype(jnp.bfloat16)

        def rs_desc(g, s, comm, sub, dev):
            return pltpu.make_async_remote_copy(
                src_ref=comm[g].at[s % 2], dst_ref=comm[g].at[(s + 1) % 2],
                send_sem=rs_send.at[s, sub, g],
                recv_sem=rs_recv.at[s, sub, g],
                device_id=(dev,), device_id_type=pl.DeviceIdType.MESH,
            )

        def ag_desc(g, h, comm, sub, dev):
            return pltpu.make_async_remote_copy(
                src_ref=comm[g].at[(h + 1) % 2], dst_ref=comm[g].at[h % 2],
                send_sem=ag_send.at[h, sub, g],
                recv_sem=ag_recv.at[h, sub, g],
                device_id=(dev,), device_id_type=pl.DeviceIdType.MESH,
            )

        rsR = [[rs_desc(g, s, commR, 0, right) for s in range(N_DEV - 1)]
               for g in range(NG)]
        rsL = [[rs_desc(g, s, commL, 1, left) for s in range(N_DEV - 1)]
               for g in range(NG)]
        agR = [[ag_desc(g, h, commR, 0, right) for h in range(N_DEV - 1)]
               for g in range(NG)]
        agL = [[ag_desc(g, h, commL, 1, left) for h in range(N_DEV - 1)]
               for g in range(NG)]

        copies = []

        def store_half(comm, g, slot, c, col0, sem_idx):
            cp = pltpu.make_async_copy(
                comm[g].at[slot],
                out_ref.at[pl.ds(c * CH + g * GR, GR), pl.ds(col0, H)],
                copy_sems.at[sem_idx],
            )
            cp.start()
            copies.append(cp)

        for g in range(NG):
            commR[g][0] = mm_half(my, g, 0)
            rsR[g][0].start()
            commL[g][0] = mm_half(my, g, H)
            rsL[g][0].start()

        precompute(0, lax.rem(my - 1 + N_DEV, N_DEV), lax.rem(my + 1, N_DEV))

        for s in range(N_DEV - 1):
            last = s == N_DEV - 2
            for g in range(NG):
                rsR[g][s].wait()
                add_staged(g, commR[g], (s + 1) % 2, 0)
                if not last:
                    rsR[g][s + 1].start()
                else:
                    agR[g][0].start()
                    store_half(commR, g, 1, lax.rem(my + 1, N_DEV), 0,
                               g * 2)
                rsL[g][s].wait()
                add_staged(g, commL[g], (s + 1) % 2, H)
                if not last:
                    rsL[g][s + 1].start()
                else:
                    agL[g][0].start()
                    store_half(commL, g, 1, lax.rem(my + 3, N_DEV), H,
                               g * 2 + 1)
                if g == 0:
                    precompute(1, lax.rem(my - s - 1 + N_DEV, N_DEV),
                               lax.rem(my + s + 1, N_DEV))
                elif not last:
                    precompute(0, lax.rem(my - s - 2 + N_DEV, N_DEV),
                               lax.rem(my + s + 2, N_DEV))

        for h in range(N_DEV - 1):
            for g in range(NG):
                agR[g][h].wait()
                if h < N_DEV - 2:
                    agR[g][h + 1].start()
                store_half(commR, g, h % 2, lax.rem(my - h + N_DEV, N_DEV),
                           0, 4 + h * 4 + g * 2)
                agL[g][h].wait()
                if h < N_DEV - 2:
                    agL[g][h + 1].start()
                store_half(commL, g, h % 2, lax.rem(my + h, N_DEV),
                           H, 4 + h * 4 + g * 2 + 1)

        for cp in copies:
            cp.wait()

    return pl.pallas_call(
        body,
        out_shape=jax.ShapeDtypeStruct((M, N), jnp.bfloat16),
        in_specs=[
            pl.BlockSpec(memory_space=pltpu.VMEM),
            pl.BlockSpec(memory_space=pltpu.VMEM),
        ],
        out_specs=pl.BlockSpec(memory_space=pl.ANY),
        scratch_shapes=[
            pltpu.VMEM((2, GR, H), jnp.bfloat16),
            pltpu.VMEM((2, GR, H), jnp.bfloat16),
            pltpu.VMEM((2, GR, H), jnp.bfloat16),
            pltpu.VMEM((2, GR, H), jnp.bfloat16),
            pltpu.VMEM((GR, N), jnp.bfloat16),
            pltpu.VMEM((GR, N), jnp.bfloat16),
            pltpu.SemaphoreType.DMA((N_DEV - 1, 2, NG)),
            pltpu.SemaphoreType.DMA((N_DEV - 1, 2, NG)),
            pltpu.SemaphoreType.DMA((N_DEV - 1, 2, NG)),
            pltpu.SemaphoreType.DMA((N_DEV - 1, 2, NG)),
            pltpu.SemaphoreType.DMA((16,)),
        ],
        compiler_params=pltpu.CompilerParams(
            collective_id=0,
            vmem_limit_bytes=62 * 1024 * 1024,
        ),
    )(A16, B16)
